# Initial kernel scaffold; baseline (speedup 1.0000x reference)
import jax
import jax.numpy as jnp
from jax.experimental import pallas as pl


def kernel(feat, edge_index_cites, edge_index_refs, W_cites, b_cites, W_refs, b_refs):
    raise NotImplementedError("write your pallas kernel here")


# speedup vs baseline: 2.3098x; 2.3098x over previous
import jax
import jax.numpy as jnp
from jax import lax
from jax.experimental import pallas as pl
from jax.experimental.pallas import tpu as pltpu
from jax.experimental.pallas import tpu_sc as plsc

N_NODES = 10000
D = 128
N_EDGES = 160000

N_TILES = 16
CHUNK = 128
CHUNKS_PER_TILE = 80
EDGES_PAD = N_TILES * CHUNKS_PER_TILE * CHUNK
NPAD = 10240
STRIPE = NPAD // N_TILES
JUNK_ROW = N_NODES
DEG_ROWS = 128


def _sc_scatter(x_hbm, src_hbm, dst_hbm, id_hbm, zrows_hbm,
                agg_out, deg_out,
                src_v, dst_v, dhi_v, dlo_v, rows_v, ones_v, acc_sh, deg_sh, sem):
    c = lax.axis_index("c")
    s = lax.axis_index("s")
    r0 = s * STRIPE

    pltpu.sync_copy(zrows_hbm, rows_v)
    for k in range(STRIPE // CHUNK):
        pltpu.sync_copy(rows_v, acc_sh.at[pl.ds(r0 + k * CHUNK, CHUNK), :])

    @pl.when(s == 0)
    def _():
        pltpu.sync_copy(rows_v, deg_sh)

    plsc.subcore_barrier()

    def body(j, carry):
        pltpu.sync_copy(src_hbm.at[c, s, j], src_v)
        pltpu.sync_copy(dst_hbm.at[c, s, j], dst_v)
        for t in range(CHUNK // 16):
            dd = dst_v[pl.ds(t * 16, 16)]
            dhi_v[pl.ds(t * 16, 16)] = lax.shift_right_logical(dd, 7)
            dlo_v[pl.ds(t * 16, 16)] = lax.bitwise_and(dd, 127)
        pltpu.async_copy(x_hbm.at[src_v], rows_v, sem).wait()
        pltpu.sync_copy(rows_v, acc_sh.at[dst_v], add=True)
        pltpu.async_copy(id_hbm.at[dlo_v], ones_v, sem).wait()
        pltpu.sync_copy(ones_v, deg_sh.at[dhi_v], add=True)
        return carry

    lax.fori_loop(0, CHUNKS_PER_TILE, body, 0)
    plsc.subcore_barrier()

    for k in range(STRIPE // CHUNK):
        rk = r0 + k * CHUNK
        pltpu.sync_copy(acc_sh.at[pl.ds(rk, CHUNK), :], rows_v)
        pltpu.sync_copy(rows_v, agg_out.at[c, pl.ds(rk, CHUNK), :])

    @pl.when(s == 0)
    def _():
        pltpu.sync_copy(deg_sh, ones_v)
        pltpu.sync_copy(ones_v, deg_out.at[c])


BR = 2048


def _tc_body(agg_ref, deg_ref, wc_ref, wr_ref, bc_ref, br_ref, out_ref):
    a = agg_ref[...]
    dg = deg_ref[...]

    def norm(ai, di):
        ai3 = ai.reshape(BR // 128, 128, D)
        di3 = jnp.maximum(di, 1.0)[:, :, None]
        return (ai3 / di3).reshape(BR, D)

    hc = norm(a[0], dg[0])
    hr = norm(a[1], dg[1])
    out_ref[...] = (
        jnp.dot(hc, wc_ref[...], preferred_element_type=jnp.float32)
        + jnp.dot(hr, wr_ref[...], preferred_element_type=jnp.float32)
        + bc_ref[...] + br_ref[...]
    )


def kernel(feat, edge_index_cites, edge_index_refs, W_cites, b_cites, W_refs, b_refs):
    pad = EDGES_PAD - N_EDGES
    src_pad = jnp.zeros((pad,), jnp.int32)
    dst_pad = jnp.full((pad,), JUNK_ROW, jnp.int32)
    src_all = jnp.stack([
        jnp.concatenate([edge_index_cites[0], src_pad]),
        jnp.concatenate([edge_index_refs[0], src_pad]),
    ]).reshape(2, N_TILES, CHUNKS_PER_TILE, CHUNK)
    dst_all = jnp.stack([
        jnp.concatenate([edge_index_cites[1], dst_pad]),
        jnp.concatenate([edge_index_refs[1], dst_pad]),
    ]).reshape(2, N_TILES, CHUNKS_PER_TILE, CHUNK)

    ident = jnp.eye(128, dtype=jnp.float32)
    zrows = jnp.zeros((CHUNK, D), jnp.float32)

    mesh = plsc.VectorSubcoreMesh(core_axis_name="c", subcore_axis_name="s")
    agg, deg = pl.kernel(
        _sc_scatter,
        out_type=[
            jax.ShapeDtypeStruct((2, NPAD, D), jnp.float32),
            jax.ShapeDtypeStruct((2, DEG_ROWS, 128), jnp.float32),
        ],
        scratch_types=[
            pltpu.VMEM((CHUNK,), jnp.int32),
            pltpu.VMEM((CHUNK,), jnp.int32),
            pltpu.VMEM((CHUNK,), jnp.int32),
            pltpu.VMEM((CHUNK,), jnp.int32),
            pltpu.VMEM((CHUNK, D), jnp.float32),
            pltpu.VMEM((CHUNK, 128), jnp.float32),
            pltpu.VMEM_SHARED((NPAD, D), jnp.float32),
            pltpu.VMEM_SHARED((DEG_ROWS, 128), jnp.float32),
            pltpu.SemaphoreType.DMA,
        ],
        mesh=mesh,
    )(feat, src_all, dst_all, ident, zrows)

    out = pl.pallas_call(
        _tc_body,
        grid=(NPAD // BR,),
        in_specs=[
            pl.BlockSpec((2, BR, D), lambda i: (0, i, 0)),
            pl.BlockSpec((2, BR // 128, 128), lambda i: (0, i, 0)),
            pl.BlockSpec((D, D), lambda i: (0, 0)),
            pl.BlockSpec((D, D), lambda i: (0, 0)),
            pl.BlockSpec((1, D), lambda i: (0, 0)),
            pl.BlockSpec((1, D), lambda i: (0, 0)),
        ],
        out_specs=pl.BlockSpec((BR, D), lambda i: (i, 0)),
        out_shape=jax.ShapeDtypeStruct((N_NODES, D), jnp.float32),
    )(agg, deg, W_cites, W_refs,
      b_cites.reshape(1, D), b_refs.reshape(1, D))
    return out


# speedup vs baseline: 4.1324x; 1.7891x over previous
import jax
import jax.numpy as jnp
from jax import lax
from jax.experimental import pallas as pl
from jax.experimental.pallas import tpu as pltpu
from jax.experimental.pallas import tpu_sc as plsc

N_NODES = 10000
D = 128
N_EDGES = 160000

N_TILES = 16
CHUNK = 128
SB = 8
N_SB = 10
CHUNKS_PER_TILE = SB * N_SB
EDGES_PAD = N_TILES * CHUNKS_PER_TILE * CHUNK
NPAD = 10240
STRIPE = NPAD // N_TILES
JUNK_ROW = N_NODES

DEG_HI = NPAD // 128
E_SUB = 1024
E_COLS = 8
E_STEP = E_SUB * E_COLS


def _sc_scatter(x_hbm, src_hbm, dst_hbm, zrows_hbm,
                agg_out,
                src_v, dst_v, rows0_v, rows1_v, acc_sh, sem0, sem1):
    c = lax.axis_index("c")
    s = lax.axis_index("s")
    r0 = s * STRIPE

    pltpu.sync_copy(zrows_hbm, rows0_v)
    for k in range(STRIPE // CHUNK):
        pltpu.sync_copy(rows0_v, acc_sh.at[pl.ds(r0 + k * CHUNK, CHUNK), :])
    plsc.subcore_barrier()

    bufs = (rows0_v, rows1_v)
    sems = (sem0, sem1)

    def body(g, carry):
        pltpu.sync_copy(src_hbm.at[c, s, pl.ds(g * SB, SB)], src_v)
        pltpu.sync_copy(dst_hbm.at[c, s, pl.ds(g * SB, SB)], dst_v)
        pend = pltpu.async_copy(x_hbm.at[src_v.at[0]], bufs[0], sems[0])
        for j in range(SB):
            pend.wait()
            if j + 1 < SB:
                pend = pltpu.async_copy(
                    x_hbm.at[src_v.at[j + 1]], bufs[(j + 1) % 2], sems[(j + 1) % 2])
            pltpu.sync_copy(bufs[j % 2], acc_sh.at[dst_v.at[j]], add=True)
        return carry

    lax.fori_loop(0, N_SB, body, 0)
    plsc.subcore_barrier()

    for k in range(STRIPE // CHUNK):
        rk = r0 + k * CHUNK
        pltpu.sync_copy(acc_sh.at[pl.ds(rk, CHUNK), :], rows0_v)
        pltpu.sync_copy(rows0_v, agg_out.at[c, pl.ds(rk, CHUNK), :])


def _deg_body(dst_ref, deg_ref):
    @pl.when(pl.program_id(1) == 0)
    def _():
        deg_ref[...] = jnp.zeros_like(deg_ref)

    dd = dst_ref[...].reshape(E_SUB, E_COLS)
    acc = jnp.zeros((DEG_HI, 128), jnp.float32)
    for t in range(E_COLS):
        d = dd[:, t:t + 1]
        hi = lax.shift_right_logical(d, 7)
        lo = lax.bitwise_and(d, 127)
        oh_hi = (hi == lax.broadcasted_iota(jnp.int32, (1, DEG_HI), 1)).astype(jnp.float32)
        oh_lo = (lo == lax.broadcasted_iota(jnp.int32, (1, 128), 1)).astype(jnp.float32)
        acc = acc + lax.dot_general(oh_hi, oh_lo, (((0,), (0,)), ((), ())),
                                    preferred_element_type=jnp.float32)
    deg_ref[...] = deg_ref[...] + acc[None]


BR = 2048


def _tc_body(agg_ref, deg_ref, wc_ref, wr_ref, bc_ref, br_ref, out_ref):
    a = agg_ref[...]
    dg = deg_ref[...]

    def norm(ai, di):
        ai3 = ai.reshape(BR // 128, 128, D)
        di3 = jnp.maximum(di, 1.0)[:, :, None]
        return (ai3 / di3).reshape(BR, D)

    hc = norm(a[0], dg[0])
    hr = norm(a[1], dg[1])
    out_ref[...] = (
        jnp.dot(hc, wc_ref[...], preferred_element_type=jnp.float32)
        + jnp.dot(hr, wr_ref[...], preferred_element_type=jnp.float32)
        + bc_ref[...] + br_ref[...]
    )


def kernel(feat, edge_index_cites, edge_index_refs, W_cites, b_cites, W_refs, b_refs):
    pad = EDGES_PAD - N_EDGES
    src_pad = jnp.zeros((pad,), jnp.int32)
    dst_pad = jnp.full((pad,), JUNK_ROW, jnp.int32)
    src_all = jnp.stack([
        jnp.concatenate([edge_index_cites[0], src_pad]),
        jnp.concatenate([edge_index_refs[0], src_pad]),
    ]).reshape(2, N_TILES, CHUNKS_PER_TILE, CHUNK)
    dst_flat = jnp.stack([
        jnp.concatenate([edge_index_cites[1], dst_pad]),
        jnp.concatenate([edge_index_refs[1], dst_pad]),
    ])
    dst_all = dst_flat.reshape(2, N_TILES, CHUNKS_PER_TILE, CHUNK)

    zrows = jnp.zeros((CHUNK, D), jnp.float32)

    mesh = plsc.VectorSubcoreMesh(core_axis_name="c", subcore_axis_name="s")
    agg = pl.kernel(
        _sc_scatter,
        out_type=[jax.ShapeDtypeStruct((2, NPAD, D), jnp.float32)],
        scratch_types=[
            pltpu.VMEM((SB, CHUNK), jnp.int32),
            pltpu.VMEM((SB, CHUNK), jnp.int32),
            pltpu.VMEM((CHUNK, D), jnp.float32),
            pltpu.VMEM((CHUNK, D), jnp.float32),
            pltpu.VMEM_SHARED((NPAD, D), jnp.float32),
            pltpu.SemaphoreType.DMA,
            pltpu.SemaphoreType.DMA,
        ],
        mesh=mesh,
    )(feat, src_all, dst_all, zrows)[0]

    dst_steps = dst_flat.reshape(2, EDGES_PAD // E_STEP, E_SUB, E_COLS)
    deg = pl.pallas_call(
        _deg_body,
        grid=(2, EDGES_PAD // E_STEP),
        in_specs=[pl.BlockSpec((1, 1, E_SUB, E_COLS), lambda e, i: (e, i, 0, 0))],
        out_specs=pl.BlockSpec((1, DEG_HI, 128), lambda e, i: (e, 0, 0)),
        out_shape=jax.ShapeDtypeStruct((2, DEG_HI, 128), jnp.float32),
    )(dst_steps)

    out = pl.pallas_call(
        _tc_body,
        grid=(NPAD // BR,),
        in_specs=[
            pl.BlockSpec((2, BR, D), lambda i: (0, i, 0)),
            pl.BlockSpec((2, BR // 128, 128), lambda i: (0, i, 0)),
            pl.BlockSpec((D, D), lambda i: (0, 0)),
            pl.BlockSpec((D, D), lambda i: (0, 0)),
            pl.BlockSpec((1, D), lambda i: (0, 0)),
            pl.BlockSpec((1, D), lambda i: (0, 0)),
        ],
        out_specs=pl.BlockSpec((BR, D), lambda i: (i, 0)),
        out_shape=jax.ShapeDtypeStruct((N_NODES, D), jnp.float32),
    )(agg, deg, W_cites, W_refs,
      b_cites.reshape(1, D), b_refs.reshape(1, D))
    return out


# speedup vs baseline: 4.2352x; 1.0249x over previous
import jax
import jax.numpy as jnp
from jax import lax
from jax.experimental import pallas as pl
from jax.experimental.pallas import tpu as pltpu
from jax.experimental.pallas import tpu_sc as plsc

N_NODES = 10000
D = 128
N_EDGES = 160000

N_TILES = 16
CHUNK = 128
SB = 8
N_SB = 10
CHUNKS_PER_TILE = SB * N_SB
EDGES_PAD = N_TILES * CHUNKS_PER_TILE * CHUNK
NPAD = 10240
STRIPE = NPAD // N_TILES
JUNK_ROW = N_NODES

DEG_HI = NPAD // 128
E_SUB = 1024
E_COLS = 16
E_STEP = E_SUB * E_COLS


def _sc_scatter(x_hbm, src_hbm, dst_hbm, zrows_hbm,
                agg_out,
                src_v, dst_v, rows0_v, rows1_v, acc_sh, sem0, sem1):
    c = lax.axis_index("c")
    s = lax.axis_index("s")
    r0 = s * STRIPE

    pltpu.sync_copy(zrows_hbm, rows0_v)
    for k in range(STRIPE // CHUNK):
        pltpu.sync_copy(rows0_v, acc_sh.at[pl.ds(r0 + k * CHUNK, CHUNK), :])
    plsc.subcore_barrier()

    bufs = (rows0_v, rows1_v)
    sems = (sem0, sem1)

    def body(g, carry):
        pltpu.sync_copy(src_hbm.at[c, s, pl.ds(g * SB, SB)], src_v)
        pltpu.sync_copy(dst_hbm.at[c, s, pl.ds(g * SB, SB)], dst_v)
        pend = pltpu.async_copy(x_hbm.at[src_v.at[0]], bufs[0], sems[0])
        for j in range(SB):
            pend.wait()
            if j + 1 < SB:
                pend = pltpu.async_copy(
                    x_hbm.at[src_v.at[j + 1]], bufs[(j + 1) % 2], sems[(j + 1) % 2])
            pltpu.sync_copy(bufs[j % 2], acc_sh.at[dst_v.at[j]], add=True)
        return carry

    lax.fori_loop(0, N_SB, body, 0)
    plsc.subcore_barrier()

    for k in range(STRIPE // CHUNK):
        rk = r0 + k * CHUNK
        pltpu.sync_copy(acc_sh.at[pl.ds(rk, CHUNK), :], rows0_v)
        pltpu.sync_copy(rows0_v, agg_out.at[c, pl.ds(rk, CHUNK), :])


def _deg_body(dst_ref, deg_ref):
    @pl.when(pl.program_id(1) == 0)
    def _():
        deg_ref[...] = jnp.zeros_like(deg_ref)

    dd = dst_ref[...].reshape(E_SUB, E_COLS)
    acc = jnp.zeros((DEG_HI, 128), jnp.float32)
    for t in range(E_COLS):
        d = dd[:, t:t + 1]
        hi = lax.shift_right_logical(d, 7)
        lo = lax.bitwise_and(d, 127)
        oh_hi = (hi == lax.broadcasted_iota(jnp.int32, (1, DEG_HI), 1)).astype(jnp.float32)
        oh_lo = (lo == lax.broadcasted_iota(jnp.int32, (1, 128), 1)).astype(jnp.float32)
        acc = acc + lax.dot_general(oh_hi, oh_lo, (((0,), (0,)), ((), ())),
                                    preferred_element_type=jnp.float32)
    deg_ref[...] = deg_ref[...] + acc[None]


BR = 2048


def _tc_body(agg_ref, deg_ref, wc_ref, wr_ref, bc_ref, br_ref, out_ref):
    a = agg_ref[...]
    dg = deg_ref[...]

    def norm(ai, di):
        ai3 = ai.reshape(BR // 128, 128, D)
        di3 = jnp.maximum(di, 1.0)[:, :, None]
        return (ai3 / di3).reshape(BR, D)

    hc = norm(a[0], dg[0])
    hr = norm(a[1], dg[1])
    out_ref[...] = (
        jnp.dot(hc, wc_ref[...], preferred_element_type=jnp.float32)
        + jnp.dot(hr, wr_ref[...], preferred_element_type=jnp.float32)
        + bc_ref[...] + br_ref[...]
    )


def kernel(feat, edge_index_cites, edge_index_refs, W_cites, b_cites, W_refs, b_refs):
    pad = EDGES_PAD - N_EDGES
    src_pad = jnp.zeros((pad,), jnp.int32)
    dst_pad = jnp.full((pad,), JUNK_ROW, jnp.int32)
    src_all = jnp.stack([
        jnp.concatenate([edge_index_cites[0], src_pad]),
        jnp.concatenate([edge_index_refs[0], src_pad]),
    ]).reshape(2, N_TILES, CHUNKS_PER_TILE, CHUNK)
    dst_flat = jnp.stack([
        jnp.concatenate([edge_index_cites[1], dst_pad]),
        jnp.concatenate([edge_index_refs[1], dst_pad]),
    ])
    dst_all = dst_flat.reshape(2, N_TILES, CHUNKS_PER_TILE, CHUNK)

    zrows = jnp.zeros((CHUNK, D), jnp.float32)

    mesh = plsc.VectorSubcoreMesh(core_axis_name="c", subcore_axis_name="s")
    agg = pl.kernel(
        _sc_scatter,
        out_type=[jax.ShapeDtypeStruct((2, NPAD, D), jnp.float32)],
        scratch_types=[
            pltpu.VMEM((SB, CHUNK), jnp.int32),
            pltpu.VMEM((SB, CHUNK), jnp.int32),
            pltpu.VMEM((CHUNK, D), jnp.float32),
            pltpu.VMEM((CHUNK, D), jnp.float32),
            pltpu.VMEM_SHARED((NPAD, D), jnp.float32),
            pltpu.SemaphoreType.DMA,
            pltpu.SemaphoreType.DMA,
        ],
        mesh=mesh,
    )(feat, src_all, dst_all, zrows)[0]

    dst_steps = dst_flat.reshape(2, EDGES_PAD // E_STEP, E_SUB, E_COLS)
    deg = pl.pallas_call(
        _deg_body,
        grid=(2, EDGES_PAD // E_STEP),
        in_specs=[pl.BlockSpec((1, 1, E_SUB, E_COLS), lambda e, i: (e, i, 0, 0))],
        out_specs=pl.BlockSpec((1, DEG_HI, 128), lambda e, i: (e, 0, 0)),
        out_shape=jax.ShapeDtypeStruct((2, DEG_HI, 128), jnp.float32),
    )(dst_steps)

    out = pl.pallas_call(
        _tc_body,
        grid=(NPAD // BR,),
        in_specs=[
            pl.BlockSpec((2, BR, D), lambda i: (0, i, 0)),
            pl.BlockSpec((2, BR // 128, 128), lambda i: (0, i, 0)),
            pl.BlockSpec((D, D), lambda i: (0, 0)),
            pl.BlockSpec((D, D), lambda i: (0, 0)),
            pl.BlockSpec((1, D), lambda i: (0, 0)),
            pl.BlockSpec((1, D), lambda i: (0, 0)),
        ],
        out_specs=pl.BlockSpec((BR, D), lambda i: (i, 0)),
        out_shape=jax.ShapeDtypeStruct((N_NODES, D), jnp.float32),
    )(agg, deg, W_cites, W_refs,
      b_cites.reshape(1, D), b_refs.reshape(1, D))
    return out


# speedup vs baseline: 4.3032x; 1.0161x over previous
import jax
import jax.numpy as jnp
from jax import lax
from jax.experimental import pallas as pl
from jax.experimental.pallas import tpu as pltpu
from jax.experimental.pallas import tpu_sc as plsc

N_NODES = 10000
D = 128
N_EDGES = 160000

N_TILES = 16
CHUNK = 128
SB = 16
N_SB = 5
CHUNKS_PER_TILE = SB * N_SB
EDGES_PAD = N_TILES * CHUNKS_PER_TILE * CHUNK
NPAD = 10240
STRIPE = NPAD // N_TILES
JUNK_ROW = N_NODES

DEG_HI = NPAD // 128
E_SUB = 1024
E_COLS = 32
E_STEP = E_SUB * E_COLS


def _sc_scatter(x_hbm, src_hbm, dst_hbm, zrows_hbm,
                agg_out,
                src_v, dst_v, rows0_v, rows1_v, acc_sh, sem0, sem1):
    c = lax.axis_index("c")
    s = lax.axis_index("s")
    r0 = s * STRIPE

    pltpu.sync_copy(zrows_hbm, rows0_v)
    for k in range(STRIPE // CHUNK):
        pltpu.sync_copy(rows0_v, acc_sh.at[pl.ds(r0 + k * CHUNK, CHUNK), :])
    plsc.subcore_barrier()

    bufs = (rows0_v, rows1_v)
    sems = (sem0, sem1)

    def body(g, carry):
        pltpu.sync_copy(src_hbm.at[c, s, pl.ds(g * SB, SB)], src_v)
        pltpu.sync_copy(dst_hbm.at[c, s, pl.ds(g * SB, SB)], dst_v)
        pend = pltpu.async_copy(x_hbm.at[src_v.at[0]], bufs[0], sems[0])
        for j in range(SB):
            pend.wait()
            if j + 1 < SB:
                pend = pltpu.async_copy(
                    x_hbm.at[src_v.at[j + 1]], bufs[(j + 1) % 2], sems[(j + 1) % 2])
            pltpu.sync_copy(bufs[j % 2], acc_sh.at[dst_v.at[j]], add=True)
        return carry

    lax.fori_loop(0, N_SB, body, 0)
    plsc.subcore_barrier()

    for k in range(STRIPE // CHUNK):
        rk = r0 + k * CHUNK
        pltpu.sync_copy(acc_sh.at[pl.ds(rk, CHUNK), :], rows0_v)
        pltpu.sync_copy(rows0_v, agg_out.at[c, pl.ds(rk, CHUNK), :])


def _deg_body(dst_ref, deg_ref):
    @pl.when(pl.program_id(1) == 0)
    def _():
        deg_ref[...] = jnp.zeros_like(deg_ref)

    dd = dst_ref[...].reshape(E_SUB, E_COLS)
    acc = jnp.zeros((DEG_HI, 128), jnp.float32)
    for t in range(E_COLS):
        d = dd[:, t:t + 1]
        hi = lax.shift_right_logical(d, 7)
        lo = lax.bitwise_and(d, 127)
        oh_hi = (hi == lax.broadcasted_iota(jnp.int32, (1, DEG_HI), 1)).astype(jnp.float32)
        oh_lo = (lo == lax.broadcasted_iota(jnp.int32, (1, 128), 1)).astype(jnp.float32)
        acc = acc + lax.dot_general(oh_hi, oh_lo, (((0,), (0,)), ((), ())),
                                    preferred_element_type=jnp.float32)
    deg_ref[...] = deg_ref[...] + acc[None]


BR = 2048


def _tc_body(agg_ref, deg_ref, wc_ref, wr_ref, bc_ref, br_ref, out_ref):
    a = agg_ref[...]
    dg = deg_ref[...]

    def norm(ai, di):
        ai3 = ai.reshape(BR // 128, 128, D)
        di3 = jnp.maximum(di, 1.0)[:, :, None]
        return (ai3 / di3).reshape(BR, D)

    hc = norm(a[0], dg[0])
    hr = norm(a[1], dg[1])
    out_ref[...] = (
        jnp.dot(hc, wc_ref[...], preferred_element_type=jnp.float32)
        + jnp.dot(hr, wr_ref[...], preferred_element_type=jnp.float32)
        + bc_ref[...] + br_ref[...]
    )


def kernel(feat, edge_index_cites, edge_index_refs, W_cites, b_cites, W_refs, b_refs):
    pad = EDGES_PAD - N_EDGES
    src_pad = jnp.zeros((pad,), jnp.int32)
    dst_pad = jnp.full((pad,), JUNK_ROW, jnp.int32)
    src_all = jnp.stack([
        jnp.concatenate([edge_index_cites[0], src_pad]),
        jnp.concatenate([edge_index_refs[0], src_pad]),
    ]).reshape(2, N_TILES, CHUNKS_PER_TILE, CHUNK)
    dst_flat = jnp.stack([
        jnp.concatenate([edge_index_cites[1], dst_pad]),
        jnp.concatenate([edge_index_refs[1], dst_pad]),
    ])
    dst_all = dst_flat.reshape(2, N_TILES, CHUNKS_PER_TILE, CHUNK)

    zrows = jnp.zeros((CHUNK, D), jnp.float32)

    mesh = plsc.VectorSubcoreMesh(core_axis_name="c", subcore_axis_name="s")
    agg = pl.kernel(
        _sc_scatter,
        out_type=[jax.ShapeDtypeStruct((2, NPAD, D), jnp.float32)],
        scratch_types=[
            pltpu.VMEM((SB, CHUNK), jnp.int32),
            pltpu.VMEM((SB, CHUNK), jnp.int32),
            pltpu.VMEM((CHUNK, D), jnp.float32),
            pltpu.VMEM((CHUNK, D), jnp.float32),
            pltpu.VMEM_SHARED((NPAD, D), jnp.float32),
            pltpu.SemaphoreType.DMA,
            pltpu.SemaphoreType.DMA,
        ],
        mesh=mesh,
    )(feat, src_all, dst_all, zrows)[0]

    dst_steps = dst_flat.reshape(2, EDGES_PAD // E_STEP, E_SUB, E_COLS)
    deg = pl.pallas_call(
        _deg_body,
        grid=(2, EDGES_PAD // E_STEP),
        in_specs=[pl.BlockSpec((1, 1, E_SUB, E_COLS), lambda e, i: (e, i, 0, 0))],
        out_specs=pl.BlockSpec((1, DEG_HI, 128), lambda e, i: (e, 0, 0)),
        out_shape=jax.ShapeDtypeStruct((2, DEG_HI, 128), jnp.float32),
    )(dst_steps)

    out = pl.pallas_call(
        _tc_body,
        grid=(NPAD // BR,),
        in_specs=[
            pl.BlockSpec((2, BR, D), lambda i: (0, i, 0)),
            pl.BlockSpec((2, BR // 128, 128), lambda i: (0, i, 0)),
            pl.BlockSpec((D, D), lambda i: (0, 0)),
            pl.BlockSpec((D, D), lambda i: (0, 0)),
            pl.BlockSpec((1, D), lambda i: (0, 0)),
            pl.BlockSpec((1, D), lambda i: (0, 0)),
        ],
        out_specs=pl.BlockSpec((BR, D), lambda i: (i, 0)),
        out_shape=jax.ShapeDtypeStruct((N_NODES, D), jnp.float32),
    )(agg, deg, W_cites, W_refs,
      b_cites.reshape(1, D), b_refs.reshape(1, D))
    return out


# speedup vs baseline: 4.3071x; 1.0009x over previous
import jax
import jax.numpy as jnp
from jax import lax
from jax.experimental import pallas as pl
from jax.experimental.pallas import tpu as pltpu
from jax.experimental.pallas import tpu_sc as plsc

N_NODES = 10000
D = 128
N_EDGES = 160000

N_TILES = 16
CHUNK = 128
SB = 16
N_SB = 5
CHUNKS_PER_TILE = SB * N_SB
EDGES_PAD = N_TILES * CHUNKS_PER_TILE * CHUNK
NPAD = 10240
STRIPE = NPAD // N_TILES
JUNK_ROW = N_NODES

DEG_HI = NPAD // 128
E_SUB = 1024
E_COLS = 32
E_STEP = E_SUB * E_COLS


def _sc_scatter(x_hbm, src_hbm, dst_hbm, zrows_hbm,
                agg_out,
                src_v, dst_v, rows0_v, rows1_v, acc_sh, sem0, sem1):
    c = lax.axis_index("c")
    s = lax.axis_index("s")
    r0 = s * STRIPE

    pltpu.sync_copy(zrows_hbm, rows0_v)
    for k in range(STRIPE // CHUNK):
        pltpu.sync_copy(rows0_v, acc_sh.at[pl.ds(r0 + k * CHUNK, CHUNK), :])
    plsc.subcore_barrier()

    bufs = (rows0_v, rows1_v)
    sems = (sem0, sem1)

    def body(g, carry):
        pltpu.sync_copy(src_hbm.at[c, s, pl.ds(g * SB, SB)], src_v)
        pltpu.sync_copy(dst_hbm.at[c, s, pl.ds(g * SB, SB)], dst_v)
        pend = pltpu.async_copy(x_hbm.at[src_v.at[0]], bufs[0], sems[0])
        for j in range(SB):
            pend.wait()
            if j + 1 < SB:
                pend = pltpu.async_copy(
                    x_hbm.at[src_v.at[j + 1]], bufs[(j + 1) % 2], sems[(j + 1) % 2])
            pltpu.sync_copy(bufs[j % 2], acc_sh.at[dst_v.at[j]], add=True)
        return carry

    lax.fori_loop(0, N_SB, body, 0)
    plsc.subcore_barrier()

    for k in range(STRIPE // CHUNK):
        rk = r0 + k * CHUNK
        pltpu.sync_copy(acc_sh.at[pl.ds(rk, CHUNK), :], rows0_v)
        pltpu.sync_copy(rows0_v, agg_out.at[c, pl.ds(rk, CHUNK), :])


def _deg_body(dst_ref, deg_ref):
    @pl.when(pl.program_id(1) == 0)
    def _():
        deg_ref[...] = jnp.zeros_like(deg_ref)

    dd = dst_ref[...].reshape(E_SUB, E_COLS)
    acc = jnp.zeros((DEG_HI, 128), jnp.float32)
    for t in range(E_COLS):
        d = dd[:, t:t + 1]
        hi = lax.shift_right_logical(d, 7)
        lo = lax.bitwise_and(d, 127)
        oh_hi = (hi == lax.broadcasted_iota(jnp.int32, (1, DEG_HI), 1)).astype(jnp.float32)
        oh_lo = (lo == lax.broadcasted_iota(jnp.int32, (1, 128), 1)).astype(jnp.float32)
        acc = acc + lax.dot_general(oh_hi, oh_lo, (((0,), (0,)), ((), ())),
                                    preferred_element_type=jnp.float32)
    deg_ref[...] = deg_ref[...] + acc[None]


BR = 5120


def _tc_body(agg_ref, deg_ref, wc_ref, wr_ref, bc_ref, br_ref, out_ref):
    a = agg_ref[...]
    dg = deg_ref[...]

    def norm(ai, di):
        ai3 = ai.reshape(BR // 128, 128, D)
        di3 = jnp.maximum(di, 1.0)[:, :, None]
        return (ai3 / di3).reshape(BR, D)

    hc = norm(a[0], dg[0])
    hr = norm(a[1], dg[1])
    out_ref[...] = (
        jnp.dot(hc, wc_ref[...], preferred_element_type=jnp.float32)
        + jnp.dot(hr, wr_ref[...], preferred_element_type=jnp.float32)
        + bc_ref[...] + br_ref[...]
    )


def kernel(feat, edge_index_cites, edge_index_refs, W_cites, b_cites, W_refs, b_refs):
    pad = EDGES_PAD - N_EDGES
    src_pad = jnp.zeros((pad,), jnp.int32)
    dst_pad = jnp.full((pad,), JUNK_ROW, jnp.int32)
    src_all = jnp.stack([
        jnp.concatenate([edge_index_cites[0], src_pad]),
        jnp.concatenate([edge_index_refs[0], src_pad]),
    ]).reshape(2, N_TILES, CHUNKS_PER_TILE, CHUNK)
    dst_flat = jnp.stack([
        jnp.concatenate([edge_index_cites[1], dst_pad]),
        jnp.concatenate([edge_index_refs[1], dst_pad]),
    ])
    dst_all = dst_flat.reshape(2, N_TILES, CHUNKS_PER_TILE, CHUNK)

    zrows = jnp.zeros((CHUNK, D), jnp.float32)

    mesh = plsc.VectorSubcoreMesh(core_axis_name="c", subcore_axis_name="s")
    agg = pl.kernel(
        _sc_scatter,
        out_type=[jax.ShapeDtypeStruct((2, NPAD, D), jnp.float32)],
        scratch_types=[
            pltpu.VMEM((SB, CHUNK), jnp.int32),
            pltpu.VMEM((SB, CHUNK), jnp.int32),
            pltpu.VMEM((CHUNK, D), jnp.float32),
            pltpu.VMEM((CHUNK, D), jnp.float32),
            pltpu.VMEM_SHARED((NPAD, D), jnp.float32),
            pltpu.SemaphoreType.DMA,
            pltpu.SemaphoreType.DMA,
        ],
        mesh=mesh,
    )(feat, src_all, dst_all, zrows)[0]

    dst_steps = dst_flat.reshape(2, EDGES_PAD // E_STEP, E_SUB, E_COLS)
    deg = pl.pallas_call(
        _deg_body,
        grid=(2, EDGES_PAD // E_STEP),
        in_specs=[pl.BlockSpec((1, 1, E_SUB, E_COLS), lambda e, i: (e, i, 0, 0))],
        out_specs=pl.BlockSpec((1, DEG_HI, 128), lambda e, i: (e, 0, 0)),
        out_shape=jax.ShapeDtypeStruct((2, DEG_HI, 128), jnp.float32),
    )(dst_steps)

    out = pl.pallas_call(
        _tc_body,
        grid=(NPAD // BR,),
        in_specs=[
            pl.BlockSpec((2, BR, D), lambda i: (0, i, 0)),
            pl.BlockSpec((2, BR // 128, 128), lambda i: (0, i, 0)),
            pl.BlockSpec((D, D), lambda i: (0, 0)),
            pl.BlockSpec((D, D), lambda i: (0, 0)),
            pl.BlockSpec((1, D), lambda i: (0, 0)),
            pl.BlockSpec((1, D), lambda i: (0, 0)),
        ],
        out_specs=pl.BlockSpec((BR, D), lambda i: (i, 0)),
        out_shape=jax.ShapeDtypeStruct((N_NODES, D), jnp.float32),
    )(agg, deg, W_cites, W_refs,
      b_cites.reshape(1, D), b_refs.reshape(1, D))
    return out
